# baseline (speedup 1.0000x reference)
import functools

import jax
import jax.numpy as jnp
import numpy as np
from jax import lax
from jax.experimental import pallas as pl
from jax.experimental.pallas import tpu as pltpu
from jax.experimental.pallas import tpu_sc as plsc

_K = 100
_NITER = 10
_IDX_CHUNK = 128

_NPOLY = 9
_CHEB_ANG = (2 * np.arange(_NPOLY) + 1) * np.pi / (2 * _NPOLY)
_CHEB_T = 0.5 - 0.5 * np.cos(_CHEB_ANG)
_NODAL_TO_CHEB = np.stack([
    (1.0 if k == 0 else 2.0) / _NPOLY * np.cos(k * (np.pi - _CHEB_ANG))
    for k in range(_NPOLY)])


def _gather_rows(codebook, codes):
    n, d = codes.shape[0], codebook.shape[1]
    info = plsc.get_sparse_core_info()
    nw = info.num_cores * info.num_subcores
    b_per_w = n // nw
    mesh = plsc.VectorSubcoreMesh(core_axis_name="c", subcore_axis_name="s")

    @functools.partial(
        pl.kernel, mesh=mesh,
        out_type=jax.ShapeDtypeStruct((n, d), jnp.float32),
        scratch_types=[
            pltpu.VMEM((b_per_w,), jnp.int32),
            pltpu.VMEM((b_per_w, d), jnp.float32),
            pltpu.SemaphoreType.DMA,
        ],
    )
    def gather(table_hbm, idx_hbm, out_hbm, idx_v, rows_v, sem):
        wid = lax.axis_index("s") * info.num_cores + lax.axis_index("c")
        base = wid * b_per_w
        pltpu.sync_copy(idx_hbm.at[pl.ds(base, b_per_w)], idx_v)
        copies = [
            pltpu.async_copy(
                table_hbm.at[idx_v.at[pl.ds(j * _IDX_CHUNK, _IDX_CHUNK)]],
                rows_v.at[pl.ds(j * _IDX_CHUNK, _IDX_CHUNK)],
                sem)
            for j in range(b_per_w // _IDX_CHUNK)
        ]
        for cp in copies:
            cp.wait()
        pltpu.sync_copy(rows_v, out_hbm.at[pl.ds(base, b_per_w)])

    return gather(codebook, codes)


def _matmul_phase(emb_ref, g_ref, cbt_ref, b2_ref, sq_ref, sc_ref):
    emb = emb_ref[...]
    a2 = jnp.sum(emb * emb, axis=1, keepdims=True)
    prod = jax.lax.dot_general(
        emb, cbt_ref[...], (((1,), (0,)), ((), ())),
        preferred_element_type=jnp.float32)
    sq_ref[...] = jnp.maximum(a2 + b2_ref[...] - 2.0 * prod, 0.0)
    g = g_ref[...]
    g2 = jnp.sum(g * g, axis=1, keepdims=True)
    eg = jnp.sum(emb * g, axis=1, keepdims=True)
    sc_ref[...] = jnp.maximum(a2 + g2 - 2.0 * eg, 0.0)


def _select_phase(codes_ref, sq_ref, sc_ref, out_ref, *, n_rows, n_cb):
    sq = sq_ref[...]
    m = jnp.min(sq, axis=1, keepdims=True)
    big = jnp.max(sq, axis=1, keepdims=True)

    def _bisect(_, carry):
        lo, hi, cl = carry
        mid = 0.5 * (lo + hi)
        cnt = jnp.sum(jnp.where(sq_ref[...] <= mid, 1.0, 0.0),
                      axis=1, keepdims=True)
        ge = cnt >= _K
        return (jnp.where(ge, lo, mid), jnp.where(ge, mid, hi),
                jnp.where(ge, cl, cnt))

    lo0 = m - (1e-6 * m + 1e-30)
    lo, hi, cnt_lo = jax.lax.fori_loop(
        0, _NITER, _bisect, (lo0, big, jnp.zeros_like(m)))

    dmin = jnp.sqrt(m)
    w = jnp.maximum(hi - m, 1e-30)
    winv = 1.0 / w
    fvals = [jnp.exp(dmin - jnp.sqrt(m + float(t) * w)) for t in _CHEB_T]
    coef = [sum(float(_NODAL_TO_CHEB[k, j]) * fvals[j]
                for j in range(_NPOLY)) for k in range(_NPOLY)]

    sqv = sq_ref[...]
    tx = 2.0 * ((sqv - m) * winv) - 1.0
    tx2 = 2.0 * tx
    b0, b1 = coef[_NPOLY - 1], jnp.zeros_like(m)
    for k in range(_NPOLY - 2, 0, -1):
        b0, b1 = coef[k] + tx2 * b0 - b1, b0
    poly = coef[0] + tx * b0 - b1
    mask = sqv <= lo
    e_in = jnp.sum(jnp.where(mask, poly, 0.0), axis=1, keepdims=True)
    e_thr = jnp.exp(dmin - jnp.sqrt(hi))
    e_sum = e_in + (_K - cnt_lo) * e_thr

    sq_c = sc_ref[...]
    codes = codes_ref[...]
    iota = jax.lax.broadcasted_iota(jnp.int32, (n_rows, n_cb), 1)
    amin = jnp.min(jnp.where(sqv == m, iota, n_cb), axis=1, keepdims=True)
    d_c = jnp.sqrt(sq_c)
    member = sq_c <= hi
    e_final = jnp.where(member, e_sum, e_sum - e_thr + jnp.exp(dmin - d_c))
    loss_rows = (d_c - dmin) + jnp.log(e_final)
    acc_rows = jnp.where(amin == codes, 1.0, 0.0)

    lane = jax.lax.broadcasted_iota(jnp.int32, (1, 128), 1)
    contrib = (jnp.where(lane == 0, jnp.sum(loss_rows), 0.0)
               + jnp.where(lane == 1, jnp.sum(acc_rows), 0.0)
               + jnp.where(lane == 2, float(n_rows), 0.0))
    out_ref[...] += contrib


def _body(codes_ref, emb_ref, cbt_ref, g_ref, out_ref,
          sq_a, sq_b, sc_a, sc_b, b2_ref, *, n_rows, n_cb):
    i = pl.program_id(0)
    nr = pl.num_programs(0) - 1

    @pl.when(i == 0)
    def _init():
        cbt = cbt_ref[...]
        b2_ref[...] = jnp.sum(cbt * cbt, axis=0, keepdims=True)
        out_ref[...] = jnp.zeros_like(out_ref)

    @pl.when(jnp.logical_and(i < nr, i % 2 == 0))
    def _m_even():
        _matmul_phase(emb_ref, g_ref, cbt_ref, b2_ref, sq_a, sc_a)

    @pl.when(jnp.logical_and(i < nr, i % 2 == 1))
    def _m_odd():
        _matmul_phase(emb_ref, g_ref, cbt_ref, b2_ref, sq_b, sc_b)

    sel = functools.partial(_select_phase, codes_ref, out_ref=out_ref,
                            n_rows=n_rows, n_cb=n_cb)

    @pl.when(jnp.logical_and(i > 0, i % 2 == 1))
    def _s_even():
        sel(sq_ref=sq_a, sc_ref=sc_a)

    @pl.when(jnp.logical_and(i > 0, i % 2 == 0))
    def _s_odd():
        sel(sq_ref=sq_b, sc_ref=sc_b)


def kernel(student_emb, teacher_codes, codebook):
    b, c, t_emb = student_emb.shape
    t = min(t_emb, teacher_codes.shape[1])
    emb_flat = jnp.transpose(student_emb[:, :, :t], (0, 2, 1)).reshape(-1, c)
    codes_flat = teacher_codes[:, :t].reshape(-1).astype(jnp.int32)
    n = emb_flat.shape[0]
    n_cb = codebook.shape[0]
    cbt = codebook.T

    gathered = _gather_rows(codebook, codes_flat)

    r_b = 256
    while n % r_b:
        r_b //= 2
    nr = n // r_b

    body = functools.partial(_body, n_rows=r_b, n_cb=n_cb)
    mm_idx = lambda i: (jnp.minimum(i, nr - 1), 0)
    sel_idx = lambda i: (jnp.maximum(i - 1, 0), 0)
    out = pl.pallas_call(
        body,
        grid=(nr + 1,),
        in_specs=[
            pl.BlockSpec((r_b, 1), sel_idx),
            pl.BlockSpec((r_b, c), mm_idx),
            pl.BlockSpec((c, n_cb), lambda i: (0, 0)),
            pl.BlockSpec((r_b, c), mm_idx),
        ],
        out_specs=pl.BlockSpec((1, 128), lambda i: (0, 0)),
        out_shape=jax.ShapeDtypeStruct((1, 128), jnp.float32),
        scratch_shapes=[
            pltpu.VMEM((r_b, n_cb), jnp.float32),
            pltpu.VMEM((r_b, n_cb), jnp.float32),
            pltpu.VMEM((r_b, 1), jnp.float32),
            pltpu.VMEM((r_b, 1), jnp.float32),
            pltpu.VMEM((1, n_cb), jnp.float32),
        ],
        compiler_params=pltpu.CompilerParams(
            dimension_semantics=("arbitrary",)),
    )(codes_flat.reshape(-1, 1), emb_flat, cbt, gathered)

    inv_n = 1.0 / n
    loss = out[0, 0] * inv_n
    acc = out[0, 1] * inv_n
    cic = out[0, 2] * inv_n
    return (loss, acc, acc, cic)


# speedup vs baseline: 2.4227x; 2.4227x over previous
import functools

import jax
import jax.numpy as jnp
import numpy as np
from jax import lax
from jax.experimental import pallas as pl
from jax.experimental.pallas import tpu as pltpu
from jax.experimental.pallas import tpu_sc as plsc

_K = 100
_NITER = 10
_IDX_CHUNK = 128

_NPOLY = 9
_CHEB_ANG = (2 * np.arange(_NPOLY) + 1) * np.pi / (2 * _NPOLY)
_CHEB_T = 0.5 - 0.5 * np.cos(_CHEB_ANG)
_NODAL_TO_CHEB = np.stack([
    (1.0 if k == 0 else 2.0) / _NPOLY * np.cos(k * (np.pi - _CHEB_ANG))
    for k in range(_NPOLY)])


def _gather_rows(codebook, codes):
    n, d = codes.shape[0], codebook.shape[1]
    info = plsc.get_sparse_core_info()
    nw = info.num_cores * info.num_subcores
    b_per_w = n // nw
    mesh = plsc.VectorSubcoreMesh(core_axis_name="c", subcore_axis_name="s")

    @functools.partial(
        pl.kernel, mesh=mesh,
        out_type=jax.ShapeDtypeStruct((n, d), jnp.float32),
        scratch_types=[
            pltpu.VMEM((b_per_w,), jnp.int32),
            pltpu.VMEM((b_per_w, d), jnp.float32),
            pltpu.SemaphoreType.DMA,
        ],
    )
    def gather(table_hbm, idx_hbm, out_hbm, idx_v, rows_v, sem):
        wid = lax.axis_index("s") * info.num_cores + lax.axis_index("c")
        base = wid * b_per_w
        pltpu.sync_copy(idx_hbm.at[pl.ds(base, b_per_w)], idx_v)
        copies = [
            pltpu.async_copy(
                table_hbm.at[idx_v.at[pl.ds(j * _IDX_CHUNK, _IDX_CHUNK)]],
                rows_v.at[pl.ds(j * _IDX_CHUNK, _IDX_CHUNK)],
                sem)
            for j in range(b_per_w // _IDX_CHUNK)
        ]
        for cp in copies:
            cp.wait()
        pltpu.sync_copy(rows_v, out_hbm.at[pl.ds(base, b_per_w)])

    return gather(codebook, codes)


def _matmul_phase(emb_ref, g_ref, cbt_ref, b2_ref, sq_ref, sc_ref):
    emb = emb_ref[...]
    a2 = jnp.sum(emb * emb, axis=1, keepdims=True)
    prod = jax.lax.dot_general(
        emb, cbt_ref[...], (((1,), (0,)), ((), ())),
        preferred_element_type=jnp.float32)
    sq_ref[...] = jnp.maximum(a2 + b2_ref[...] - 2.0 * prod, 0.0)
    g = g_ref[...]
    g2 = jnp.sum(g * g, axis=1, keepdims=True)
    eg = jnp.sum(emb * g, axis=1, keepdims=True)
    sc_ref[...] = jnp.maximum(a2 + g2 - 2.0 * eg, 0.0)


def _select_phase(codes_ref, sq_ref, sc_ref, out_ref, *, n_rows, n_cb):
    sq = sq_ref[...]
    m = jnp.min(sq, axis=1, keepdims=True)
    big = jnp.max(sq, axis=1, keepdims=True)

    def _bisect(_, carry):
        lo, hi, cl = carry
        mid = 0.5 * (lo + hi)
        cnt = jnp.sum(jnp.where(sq_ref[...] <= mid, 1.0, 0.0),
                      axis=1, keepdims=True)
        ge = cnt >= _K
        return (jnp.where(ge, lo, mid), jnp.where(ge, mid, hi),
                jnp.where(ge, cl, cnt))

    lo0 = m - (1e-6 * m + 1e-30)
    lo, hi, cnt_lo = jax.lax.fori_loop(
        0, _NITER, _bisect, (lo0, big, jnp.zeros_like(m)))

    sqv = sq_ref[...]
    dmin = jnp.sqrt(m)
    mask = sqv <= lo
    e_in = jnp.sum(jnp.where(mask, jnp.exp(dmin - jnp.sqrt(sqv)), 0.0),
                   axis=1, keepdims=True)
    e_thr = jnp.exp(dmin - jnp.sqrt(hi))
    e_sum = e_in + (_K - cnt_lo) * e_thr

    sq_c = sc_ref[...]
    codes = codes_ref[...]
    iota = jax.lax.broadcasted_iota(jnp.int32, (n_rows, n_cb), 1)
    amin = jnp.min(jnp.where(sqv == m, iota, n_cb), axis=1, keepdims=True)
    d_c = jnp.sqrt(sq_c)
    member = sq_c <= hi
    e_final = jnp.where(member, e_sum, e_sum - e_thr + jnp.exp(dmin - d_c))
    loss_rows = (d_c - dmin) + jnp.log(e_final)
    acc_rows = jnp.where(amin == codes, 1.0, 0.0)

    lane = jax.lax.broadcasted_iota(jnp.int32, (1, 128), 1)
    contrib = (jnp.where(lane == 0, jnp.sum(loss_rows), 0.0)
               + jnp.where(lane == 1, jnp.sum(acc_rows), 0.0)
               + jnp.where(lane == 2, float(n_rows), 0.0))
    out_ref[...] += contrib


def _body(codes_ref, emb_ref, cbt_ref, g_ref, out_ref,
          sq_a, sq_b, sc_a, sc_b, b2_ref, *, n_rows, n_cb):
    i = pl.program_id(0)
    nr = pl.num_programs(0) - 1

    @pl.when(i == 0)
    def _init():
        cbt = cbt_ref[...]
        b2_ref[...] = jnp.sum(cbt * cbt, axis=0, keepdims=True)
        out_ref[...] = jnp.zeros_like(out_ref)

    @pl.when(jnp.logical_and(i < nr, i % 2 == 0))
    def _m_even():
        _matmul_phase(emb_ref, g_ref, cbt_ref, b2_ref, sq_a, sc_a)

    @pl.when(jnp.logical_and(i < nr, i % 2 == 1))
    def _m_odd():
        _matmul_phase(emb_ref, g_ref, cbt_ref, b2_ref, sq_b, sc_b)

    sel = functools.partial(_select_phase, codes_ref, out_ref=out_ref,
                            n_rows=n_rows, n_cb=n_cb)

    @pl.when(jnp.logical_and(i > 0, i % 2 == 1))
    def _s_even():
        sel(sq_ref=sq_a, sc_ref=sc_a)

    @pl.when(jnp.logical_and(i > 0, i % 2 == 0))
    def _s_odd():
        sel(sq_ref=sq_b, sc_ref=sc_b)


def kernel(student_emb, teacher_codes, codebook):
    b, c, t_emb = student_emb.shape
    t = min(t_emb, teacher_codes.shape[1])
    emb_flat = jnp.transpose(student_emb[:, :, :t], (0, 2, 1)).reshape(-1, c)
    codes_flat = teacher_codes[:, :t].reshape(-1).astype(jnp.int32)
    n = emb_flat.shape[0]
    n_cb = codebook.shape[0]
    cbt = codebook.T

    gathered = _gather_rows(codebook, codes_flat)

    r_b = 256
    while n % r_b:
        r_b //= 2
    nr = n // r_b

    body = functools.partial(_body, n_rows=r_b, n_cb=n_cb)
    mm_idx = lambda i: (jnp.minimum(i, nr - 1), 0)
    sel_idx = lambda i: (jnp.maximum(i - 1, 0), 0)
    out = pl.pallas_call(
        body,
        grid=(nr + 1,),
        in_specs=[
            pl.BlockSpec((r_b, 1), sel_idx),
            pl.BlockSpec((r_b, c), mm_idx),
            pl.BlockSpec((c, n_cb), lambda i: (0, 0)),
            pl.BlockSpec((r_b, c), mm_idx),
        ],
        out_specs=pl.BlockSpec((1, 128), lambda i: (0, 0)),
        out_shape=jax.ShapeDtypeStruct((1, 128), jnp.float32),
        scratch_shapes=[
            pltpu.VMEM((r_b, n_cb), jnp.float32),
            pltpu.VMEM((r_b, n_cb), jnp.float32),
            pltpu.VMEM((r_b, 1), jnp.float32),
            pltpu.VMEM((r_b, 1), jnp.float32),
            pltpu.VMEM((1, n_cb), jnp.float32),
        ],
        compiler_params=pltpu.CompilerParams(
            dimension_semantics=("arbitrary",)),
    )(codes_flat.reshape(-1, 1), emb_flat, cbt, gathered)

    inv_n = 1.0 / n
    loss = out[0, 0] * inv_n
    acc = out[0, 1] * inv_n
    cic = out[0, 2] * inv_n
    return (loss, acc, acc, cic)


# speedup vs baseline: 2.6961x; 1.1128x over previous
import functools

import jax
import jax.numpy as jnp
from jax import lax
from jax.experimental import pallas as pl
from jax.experimental.pallas import tpu as pltpu
from jax.experimental.pallas import tpu_sc as plsc

_K = 100
_NITER = 8
_IDX_CHUNK = 128


def _gather_rows(codebook, codes):
    n, d = codes.shape[0], codebook.shape[1]
    info = plsc.get_sparse_core_info()
    nw = info.num_cores * info.num_subcores
    b_per_w = n // nw
    mesh = plsc.VectorSubcoreMesh(core_axis_name="c", subcore_axis_name="s")

    @functools.partial(
        pl.kernel, mesh=mesh,
        out_type=jax.ShapeDtypeStruct((n, d), jnp.float32),
        scratch_types=[
            pltpu.VMEM((b_per_w,), jnp.int32),
            pltpu.VMEM((b_per_w, d), jnp.float32),
            pltpu.SemaphoreType.DMA,
        ],
    )
    def gather(table_hbm, idx_hbm, out_hbm, idx_v, rows_v, sem):
        wid = lax.axis_index("s") * info.num_cores + lax.axis_index("c")
        base = wid * b_per_w
        pltpu.sync_copy(idx_hbm.at[pl.ds(base, b_per_w)], idx_v)
        copies = [
            pltpu.async_copy(
                table_hbm.at[idx_v.at[pl.ds(j * _IDX_CHUNK, _IDX_CHUNK)]],
                rows_v.at[pl.ds(j * _IDX_CHUNK, _IDX_CHUNK)],
                sem)
            for j in range(b_per_w // _IDX_CHUNK)
        ]
        for cp in copies:
            cp.wait()
        pltpu.sync_copy(rows_v, out_hbm.at[pl.ds(base, b_per_w)])

    return gather(codebook, codes)


def _matmul_phase(emb_ref, g_ref, cbt_ref, b2_ref, sq_ref, sc_ref):
    emb = emb_ref[...]
    a2 = jnp.sum(emb * emb, axis=1, keepdims=True)
    prod = jax.lax.dot_general(
        emb, cbt_ref[...], (((1,), (0,)), ((), ())),
        preferred_element_type=jnp.float32)
    sq_ref[...] = jnp.maximum(a2 + b2_ref[...] - 2.0 * prod, 0.0)
    g = g_ref[...]
    g2 = jnp.sum(g * g, axis=1, keepdims=True)
    eg = jnp.sum(emb * g, axis=1, keepdims=True)
    sc_ref[...] = jnp.maximum(a2 + g2 - 2.0 * eg, 0.0)


def _select_phase(codes_ref, sq_ref, sc_ref, out_ref, *, n_rows, n_cb):
    sq = sq_ref[...]
    m = jnp.min(sq, axis=1, keepdims=True)
    big = jnp.max(sq, axis=1, keepdims=True)

    def _bisect(_, carry):
        lo, hi, cl = carry
        mid = 0.5 * (lo + hi)
        cnt = jnp.sum(jnp.where(sq_ref[...] <= mid, 1.0, 0.0),
                      axis=1, keepdims=True)
        ge = cnt >= _K
        return (jnp.where(ge, lo, mid), jnp.where(ge, mid, hi),
                jnp.where(ge, cl, cnt))

    lo0 = m - (1e-6 * m + 1e-30)
    lo, hi, cnt_lo = jax.lax.fori_loop(
        0, _NITER, _bisect, (lo0, big, jnp.zeros_like(m)))

    sqv = sq_ref[...]
    dmin = jnp.sqrt(m)
    mask = sqv <= lo
    e_in = jnp.sum(jnp.where(mask, jnp.exp(dmin - jnp.sqrt(sqv)), 0.0),
                   axis=1, keepdims=True)
    e_thr = jnp.exp(dmin - jnp.sqrt(hi))
    e_sum = e_in + (_K - cnt_lo) * e_thr

    sq_c = sc_ref[...]
    codes = codes_ref[...]
    iota = jax.lax.broadcasted_iota(jnp.int32, (n_rows, n_cb), 1)
    amin = jnp.min(jnp.where(sqv == m, iota, n_cb), axis=1, keepdims=True)
    d_c = jnp.sqrt(sq_c)
    member = sq_c <= hi
    e_final = jnp.where(member, e_sum, e_sum - e_thr + jnp.exp(dmin - d_c))
    loss_rows = (d_c - dmin) + jnp.log(e_final)
    acc_rows = jnp.where(amin == codes, 1.0, 0.0)

    lane = jax.lax.broadcasted_iota(jnp.int32, (1, 128), 1)
    contrib = (jnp.where(lane == 0, jnp.sum(loss_rows), 0.0)
               + jnp.where(lane == 1, jnp.sum(acc_rows), 0.0)
               + jnp.where(lane == 2, float(n_rows), 0.0))
    out_ref[...] += contrib


def _body(codes_ref, emb_ref, cbt_ref, g_ref, out_ref,
          sq_a, sq_b, sc_a, sc_b, b2_ref, *, n_rows, n_cb):
    i = pl.program_id(0)
    nr = pl.num_programs(0) - 1

    @pl.when(i == 0)
    def _init():
        cbt = cbt_ref[...]
        b2_ref[...] = jnp.sum(cbt * cbt, axis=0, keepdims=True)
        out_ref[...] = jnp.zeros_like(out_ref)

    @pl.when(jnp.logical_and(i < nr, i % 2 == 0))
    def _m_even():
        _matmul_phase(emb_ref, g_ref, cbt_ref, b2_ref, sq_a, sc_a)

    @pl.when(jnp.logical_and(i < nr, i % 2 == 1))
    def _m_odd():
        _matmul_phase(emb_ref, g_ref, cbt_ref, b2_ref, sq_b, sc_b)

    sel = functools.partial(_select_phase, codes_ref, out_ref=out_ref,
                            n_rows=n_rows, n_cb=n_cb)

    @pl.when(jnp.logical_and(i > 0, i % 2 == 1))
    def _s_even():
        sel(sq_ref=sq_a, sc_ref=sc_a)

    @pl.when(jnp.logical_and(i > 0, i % 2 == 0))
    def _s_odd():
        sel(sq_ref=sq_b, sc_ref=sc_b)


def kernel(student_emb, teacher_codes, codebook):
    b, c, t_emb = student_emb.shape
    t = min(t_emb, teacher_codes.shape[1])
    emb_flat = jnp.transpose(student_emb[:, :, :t], (0, 2, 1)).reshape(-1, c)
    codes_flat = teacher_codes[:, :t].reshape(-1).astype(jnp.int32)
    n = emb_flat.shape[0]
    n_cb = codebook.shape[0]
    cbt = codebook.T

    gathered = _gather_rows(codebook, codes_flat)

    r_b = 256
    while n % r_b:
        r_b //= 2
    nr = n // r_b

    body = functools.partial(_body, n_rows=r_b, n_cb=n_cb)
    mm_idx = lambda i: (jnp.minimum(i, nr - 1), 0)
    sel_idx = lambda i: (jnp.maximum(i - 1, 0), 0)
    out = pl.pallas_call(
        body,
        grid=(nr + 1,),
        in_specs=[
            pl.BlockSpec((r_b, 1), sel_idx),
            pl.BlockSpec((r_b, c), mm_idx),
            pl.BlockSpec((c, n_cb), lambda i: (0, 0)),
            pl.BlockSpec((r_b, c), mm_idx),
        ],
        out_specs=pl.BlockSpec((1, 128), lambda i: (0, 0)),
        out_shape=jax.ShapeDtypeStruct((1, 128), jnp.float32),
        scratch_shapes=[
            pltpu.VMEM((r_b, n_cb), jnp.float32),
            pltpu.VMEM((r_b, n_cb), jnp.float32),
            pltpu.VMEM((r_b, 1), jnp.float32),
            pltpu.VMEM((r_b, 1), jnp.float32),
            pltpu.VMEM((1, n_cb), jnp.float32),
        ],
        compiler_params=pltpu.CompilerParams(
            dimension_semantics=("arbitrary",)),
    )(codes_flat.reshape(-1, 1), emb_flat, cbt, gathered)

    inv_n = 1.0 / n
    loss = out[0, 0] * inv_n
    acc = out[0, 1] * inv_n
    cic = out[0, 2] * inv_n
    return (loss, acc, acc, cic)
